# Initial kernel scaffold; baseline (speedup 1.0000x reference)
import jax
import jax.numpy as jnp
from jax.experimental import pallas as pl


def kernel(x, lengths, table, W, b):
    raise NotImplementedError("write your pallas kernel here")


# speedup vs baseline: 2.5946x; 2.5946x over previous
import jax
import jax.numpy as jnp
from jax import lax
from jax.experimental import pallas as pl
from jax.experimental.pallas import tpu as pltpu
from jax.experimental.pallas import tpu_sc as plsc

B = 16384
S = 200
V = 1000000
D = 64
C = 16

NC, NS = 2, 16
NW = NC * NS
RPW = B // NW
CH = 64
NCHUNK = RPW // CH
G = 16
GW = G * S
NG = CH // G

MBLK = 8000


def _mm_body(t_ref, w_ref, o_ref):
    o_ref[...] = jnp.dot(t_ref[...], w_ref[...],
                         preferred_element_type=jnp.float32)


def _project_table(table, w_t):
    return pl.pallas_call(
        _mm_body,
        grid=(V // MBLK,),
        in_specs=[
            pl.BlockSpec((MBLK, D), lambda i: (i, 0)),
            pl.BlockSpec((D, C), lambda i: (0, 0)),
        ],
        out_specs=pl.BlockSpec((MBLK, C), lambda i: (i, 0)),
        out_shape=jax.ShapeDtypeStruct((V, C), jnp.float32),
    )(table, w_t)


def _sc_body(x_hbm, len_hbm, p_hbm, b_hbm, out_hbm,
             xv, rb0, rb1, outv, lenv, biasv, s0, s1):
    rbufs = (rb0, rb1)
    sems = (s0, s1)
    wid = lax.axis_index("c") * NS + lax.axis_index("s")
    base = wid * RPW

    pltpu.sync_copy(b_hbm, biasv)
    bias = biasv[...]

    def issue(g, q):
        off = pl.multiple_of(g * GW, 128)
        pltpu.async_copy(p_hbm.at[xv.at[pl.ds(off, GW)]], rbufs[q], sems[q])

    def drain(q):
        pltpu.make_async_copy(p_hbm.at[xv.at[pl.ds(0, GW)]],
                              rbufs[q], sems[q]).wait()

    def reduce_group(g, q):
        buf = rbufs[q]

        def rbody(j, _):
            tb = j * S
            a0 = buf[tb, :]
            a1 = buf[tb + 1, :]
            a2 = buf[tb + 2, :]
            a3 = buf[tb + 3, :]
            for t in range(4, S, 4):
                a0 = a0 + buf[tb + t, :]
                a1 = a1 + buf[tb + t + 1, :]
                a2 = a2 + buf[tb + t + 2, :]
                a3 = a3 + buf[tb + t + 3, :]
            s = (a0 + a1) + (a2 + a3)
            row = g * G + j
            outv[row, :] = s / lenv[row, :] + bias
            return 0

        lax.fori_loop(0, G, rbody, 0)

    def chunk_body(c, _):
        row0 = pl.multiple_of(base + c * CH, CH)
        pltpu.sync_copy(x_hbm.at[pl.ds(pl.multiple_of(row0 * S, 128),
                                       CH * S)], xv)
        pltpu.sync_copy(len_hbm.at[pl.ds(row0, CH)], lenv)
        issue(0, 0)
        issue(1, 1)

        def cbody(i, _):
            for q in range(2):
                g = i * 2 + q
                drain(q)
                reduce_group(g, q)

                @pl.when(g + 2 < NG)
                def _():
                    issue(g + 2, q)
            return 0

        lax.fori_loop(0, NG // 2, cbody, 0)
        pltpu.sync_copy(outv, out_hbm.at[pl.ds(row0, CH)])
        return 0

    lax.fori_loop(0, NCHUNK, chunk_body, 0)


def _pooled_logits(x_flat, len_bcast, p, b):
    mesh = plsc.VectorSubcoreMesh(core_axis_name="c", subcore_axis_name="s")
    call = pl.kernel(
        _sc_body,
        out_type=jax.ShapeDtypeStruct((B, C), jnp.float32),
        mesh=mesh,
        scratch_types=[
            pltpu.VMEM((CH * S,), jnp.int32),
            pltpu.VMEM((GW, C), jnp.float32),
            pltpu.VMEM((GW, C), jnp.float32),
            pltpu.VMEM((CH, C), jnp.float32),
            pltpu.VMEM((CH, C), jnp.float32),
            pltpu.VMEM((C,), jnp.float32),
            pltpu.SemaphoreType.DMA,
            pltpu.SemaphoreType.DMA,
        ],
        compiler_params=pltpu.CompilerParams(use_tc_tiling_on_sc=False),
    )
    return call(x_flat, len_bcast, p, b)


def kernel(x, lengths, table, W, b):
    p = _project_table(table, W.T)
    x_flat = x.reshape(B * S)
    len_bcast = jnp.broadcast_to(
        lengths.astype(jnp.float32)[:, None], (B, C))
    return _pooled_logits(x_flat, len_bcast, p, b)


# speedup vs baseline: 5.7084x; 2.2001x over previous
import jax
import jax.numpy as jnp
from jax import lax
from jax.experimental import pallas as pl
from jax.experimental.pallas import tpu as pltpu
from jax.experimental.pallas import tpu_sc as plsc

B = 16384
S = 200
V = 1000000
D = 64
C = 16

NC, NS = 2, 16
NW = NC * NS
RPW = B // NW
CH = 64
NCHUNK = RPW // CH
G = 16
GW = G * S
NG = CH // G

MBLK = 8192


def _mm_body(t_ref, w_ref, o_ref):
    res = lax.dot_general(t_ref[...], w_ref[...], (((0,), (0,)), ((), ())),
                          preferred_element_type=jnp.float32)
    res3 = res.reshape(MBLK // 8, 8, C)
    for j in range(8):
        o_ref[:, j * C:(j + 1) * C] = res3[:, j, :]


def _project_table(table_t, w_t):
    return pl.pallas_call(
        _mm_body,
        grid=((V + MBLK - 1) // MBLK,),
        in_specs=[
            pl.BlockSpec((D, MBLK), lambda i: (0, i)),
            pl.BlockSpec((D, C), lambda i: (0, 0)),
        ],
        out_specs=pl.BlockSpec((MBLK // 8, 8 * C), lambda i: (i, 0)),
        out_shape=jax.ShapeDtypeStruct((V // 8, 8 * C), jnp.float32),
    )(table_t, w_t)


def _sc_body(x_hbm, len_hbm, p_hbm, b_hbm, out_hbm,
             xv, rb0, rb1, outv, lenv, biasv, s0, s1):
    rbufs = (rb0, rb1)
    sems = (s0, s1)
    wid = lax.axis_index("c") * NS + lax.axis_index("s")
    base = wid * RPW

    pltpu.sync_copy(b_hbm, biasv)
    bias = biasv[...]

    def issue(g, q):
        off = pl.multiple_of(g * GW, 128)
        pltpu.async_copy(p_hbm.at[xv.at[pl.ds(off, GW)]], rbufs[q], sems[q])

    def drain(q):
        pltpu.make_async_copy(p_hbm.at[xv.at[pl.ds(0, GW)]],
                              rbufs[q], sems[q]).wait()

    def reduce_group(g, q):
        buf = rbufs[q]

        def rbody(j, _):
            tb = j * S
            a0 = buf[tb, :]
            a1 = buf[tb + 1, :]
            a2 = buf[tb + 2, :]
            a3 = buf[tb + 3, :]
            for t in range(4, S, 4):
                a0 = a0 + buf[tb + t, :]
                a1 = a1 + buf[tb + t + 1, :]
                a2 = a2 + buf[tb + t + 2, :]
                a3 = a3 + buf[tb + t + 3, :]
            s = (a0 + a1) + (a2 + a3)
            row = g * G + j
            outv[row, :] = s / lenv[row, :] + bias
            return 0

        lax.fori_loop(0, G, rbody, 0)

    def chunk_body(c, _):
        row0 = pl.multiple_of(base + c * CH, CH)
        pltpu.sync_copy(x_hbm.at[pl.ds(pl.multiple_of(row0 * S, 128),
                                       CH * S)], xv)
        pltpu.sync_copy(len_hbm.at[pl.ds(row0, CH)], lenv)
        issue(0, 0)
        issue(1, 1)

        def cbody(i, _):
            for q in range(2):
                g = i * 2 + q
                drain(q)
                reduce_group(g, q)

                @pl.when(g + 2 < NG)
                def _():
                    issue(g + 2, q)
            return 0

        lax.fori_loop(0, NG // 2, cbody, 0)
        pltpu.sync_copy(outv, out_hbm.at[pl.ds(row0, CH)])
        return 0

    lax.fori_loop(0, NCHUNK, chunk_body, 0)


def _pooled_logits(x_flat, len_bcast, p, b):
    mesh = plsc.VectorSubcoreMesh(core_axis_name="c", subcore_axis_name="s")
    call = pl.kernel(
        _sc_body,
        out_type=jax.ShapeDtypeStruct((B, C), jnp.float32),
        mesh=mesh,
        scratch_types=[
            pltpu.VMEM((CH * S,), jnp.int32),
            pltpu.VMEM((GW, C), jnp.float32),
            pltpu.VMEM((GW, C), jnp.float32),
            pltpu.VMEM((CH, C), jnp.float32),
            pltpu.VMEM((CH, C), jnp.float32),
            pltpu.VMEM((C,), jnp.float32),
            pltpu.SemaphoreType.DMA,
            pltpu.SemaphoreType.DMA,
        ],
        compiler_params=pltpu.CompilerParams(use_tc_tiling_on_sc=False),
    )
    return call(x_flat, len_bcast, p, b)


def kernel(x, lengths, table, W, b):
    p = _project_table(table.T, W.T).reshape(V, C)
    x_flat = x.reshape(B * S)
    len_bcast = jnp.broadcast_to(
        lengths.astype(jnp.float32)[:, None], (B, C))
    return _pooled_logits(x_flat, len_bcast, p, b)


# speedup vs baseline: 6.4371x; 1.1276x over previous
import jax
import jax.numpy as jnp
from jax import lax
from jax.experimental import pallas as pl
from jax.experimental.pallas import tpu as pltpu
from jax.experimental.pallas import tpu_sc as plsc

B = 16384
S = 200
V = 1000000
D = 64
C = 16

NC, NS = 2, 16
NW = NC * NS
RPW = B // NW
CH = 64
NCHUNK = RPW // CH
G = 16
GW = G * S
NG = CH // G

MBLK = 8192
NBLK = (V + MBLK - 1) // MBLK
VP = NBLK * MBLK


def _mm_body(t_ref, w_ref, o_ref):
    res = lax.dot_general(t_ref[...].astype(jnp.bfloat16),
                          w_ref[...].astype(jnp.bfloat16),
                          (((0,), (0,)), ((), ())),
                          preferred_element_type=jnp.float32)
    for j in range(8):
        o_ref[:, j * C:(j + 1) * C] = res[j * (MBLK // 8):(j + 1) * (MBLK // 8), :]


def _project_table(table_t, w_t):
    return pl.pallas_call(
        _mm_body,
        grid=(NBLK,),
        in_specs=[
            pl.BlockSpec((D, MBLK), lambda i: (0, i)),
            pl.BlockSpec((D, C), lambda i: (0, 0)),
        ],
        out_specs=pl.BlockSpec((MBLK // 8, 8 * C), lambda i: (i, 0)),
        out_shape=jax.ShapeDtypeStruct((VP // 8, 8 * C), jnp.float32),
    )(table_t, w_t)


def _sc_body(x_hbm, len_hbm, p_hbm, b_hbm, out_hbm,
             xv, rb0, rb1, outv, lenv, biasv, s0, s1):
    rbufs = (rb0, rb1)
    sems = (s0, s1)
    wid = lax.axis_index("c") * NS + lax.axis_index("s")
    base = wid * RPW

    pltpu.sync_copy(b_hbm, biasv)
    bias = biasv[...]

    def issue(g, q):
        off = pl.multiple_of(g * GW, 128)
        pltpu.async_copy(p_hbm.at[xv.at[pl.ds(off, GW)]], rbufs[q], sems[q])

    def drain(q):
        pltpu.make_async_copy(p_hbm.at[xv.at[pl.ds(0, GW)]],
                              rbufs[q], sems[q]).wait()

    def reduce_group(g, q):
        buf = rbufs[q]

        def rbody(j, _):
            tb = j * S
            a0 = buf[tb, :]
            a1 = buf[tb + 1, :]
            a2 = buf[tb + 2, :]
            a3 = buf[tb + 3, :]
            for t in range(4, S, 4):
                a0 = a0 + buf[tb + t, :]
                a1 = a1 + buf[tb + t + 1, :]
                a2 = a2 + buf[tb + t + 2, :]
                a3 = a3 + buf[tb + t + 3, :]
            s = (a0 + a1) + (a2 + a3)
            row = g * G + j
            outv[row, :] = s / lenv[row, :] + bias
            return 0

        lax.fori_loop(0, G, rbody, 0)

    def chunk_body(c, _):
        row0 = pl.multiple_of(base + c * CH, CH)
        pltpu.sync_copy(x_hbm.at[pl.ds(pl.multiple_of(row0 * S, 128),
                                       CH * S)], xv)
        pltpu.sync_copy(len_hbm.at[pl.ds(row0, CH)], lenv)
        issue(0, 0)
        issue(1, 1)

        def cbody(i, _):
            for q in range(2):
                g = i * 2 + q
                drain(q)
                reduce_group(g, q)

                @pl.when(g + 2 < NG)
                def _():
                    issue(g + 2, q)
            return 0

        lax.fori_loop(0, NG // 2, cbody, 0)
        pltpu.sync_copy(outv, out_hbm.at[pl.ds(row0, CH)])
        return 0

    lax.fori_loop(0, NCHUNK, chunk_body, 0)


def _pooled_logits(x_flat, len_bcast, p, b):
    mesh = plsc.VectorSubcoreMesh(core_axis_name="c", subcore_axis_name="s")
    call = pl.kernel(
        _sc_body,
        out_type=jax.ShapeDtypeStruct((B, C), jnp.float32),
        mesh=mesh,
        scratch_types=[
            pltpu.VMEM((CH * S,), jnp.int32),
            pltpu.VMEM((GW, C), jnp.float32),
            pltpu.VMEM((GW, C), jnp.float32),
            pltpu.VMEM((CH, C), jnp.float32),
            pltpu.VMEM((CH, C), jnp.float32),
            pltpu.VMEM((C,), jnp.float32),
            pltpu.SemaphoreType.DMA,
            pltpu.SemaphoreType.DMA,
        ],
        compiler_params=pltpu.CompilerParams(use_tc_tiling_on_sc=False),
    )
    return call(x_flat, len_bcast, p, b)


def kernel(x, lengths, table, W, b):
    p = _project_table(table.T, W.T).reshape(VP, C)
    x_lin = ((x >> 13) << 13) | ((x & 1023) << 3) | ((x >> 10) & 7)
    x_flat = x_lin.reshape(B * S)
    len_bcast = jnp.broadcast_to(
        lengths.astype(jnp.float32)[:, None], (B, C))
    return _pooled_logits(x_flat, len_bcast, p, b)


# speedup vs baseline: 7.2125x; 1.1205x over previous
import jax
import jax.numpy as jnp
from jax import lax
from jax.experimental import pallas as pl
from jax.experimental.pallas import tpu as pltpu
from jax.experimental.pallas import tpu_sc as plsc

B = 16384
S = 200
V = 1000000
D = 64
C = 16

NC, NS = 2, 16
NW = NC * NS
RPW = B // NW
CH = 32
NCHUNK = RPW // CH
G = 16
GW = G * S
NG = CH // G

MBLK = 8192
NBLK = (V + MBLK - 1) // MBLK
VP = NBLK * MBLK


def _mm_body(t_ref, w_ref, o_ref):
    res = lax.dot_general(t_ref[...].astype(jnp.bfloat16),
                          w_ref[...].astype(jnp.bfloat16),
                          (((0,), (0,)), ((), ())),
                          preferred_element_type=jnp.float32)
    for j in range(8):
        o_ref[:, j * C:(j + 1) * C] = res[j * (MBLK // 8):(j + 1) * (MBLK // 8), :]


def _project_table(table_t, w_t):
    return pl.pallas_call(
        _mm_body,
        grid=(NBLK,),
        in_specs=[
            pl.BlockSpec((D, MBLK), lambda i: (0, i)),
            pl.BlockSpec((D, C), lambda i: (0, 0)),
        ],
        out_specs=pl.BlockSpec((MBLK // 8, 8 * C), lambda i: (i, 0)),
        out_shape=jax.ShapeDtypeStruct((VP // 8, 8 * C), jnp.float32),
    )(table_t, w_t)


def _sc_body(x_hbm, len_hbm, p_hbm, b_hbm, out_hbm,
             xv0, xv1, rb0, rb1, outv, lenv, biasv,
             sr0, sr1, sx0, sx1):
    xvs = (xv0, xv1)
    rbufs = (rb0, rb1)
    sems = (sr0, sr1)
    xsems = (sx0, sx1)
    wid = lax.axis_index("c") * NS + lax.axis_index("s")
    base = wid * RPW

    pltpu.sync_copy(b_hbm, biasv)
    pltpu.sync_copy(len_hbm.at[pl.ds(base, RPW)], lenv)
    bias = biasv[...]

    def stage_x(c, xq):
        off = pl.multiple_of((base + c * CH) * S, 128)
        pltpu.async_copy(x_hbm.at[pl.ds(off, CH * S)], xvs[xq], xsems[xq])

    def wait_x(xq):
        pltpu.make_async_copy(x_hbm.at[pl.ds(0, CH * S)],
                              xvs[xq], xsems[xq]).wait()

    def issue(xq, g, q):
        off = pl.multiple_of(g * GW, 128)
        pltpu.async_copy(p_hbm.at[xvs[xq].at[pl.ds(off, GW)]],
                         rbufs[q], sems[q])

    def drain(q):
        pltpu.make_async_copy(p_hbm.at[xv0.at[pl.ds(0, GW)]],
                              rbufs[q], sems[q]).wait()

    def reduce_group(c, g, q):
        buf = rbufs[q]

        def rbody(j, _):
            tb = j * S
            a0 = buf[tb, :]
            a1 = buf[tb + 1, :]
            a2 = buf[tb + 2, :]
            a3 = buf[tb + 3, :]
            for t in range(4, S, 4):
                a0 = a0 + buf[tb + t, :]
                a1 = a1 + buf[tb + t + 1, :]
                a2 = a2 + buf[tb + t + 2, :]
                a3 = a3 + buf[tb + t + 3, :]
            s = (a0 + a1) + (a2 + a3)
            row = g * G + j
            outv[row, :] = s / lenv[c * CH + row, :] + bias
            return 0

        lax.fori_loop(0, G, rbody, 0)

    stage_x(0, 0)
    wait_x(0)
    issue(0, 0, 0)
    issue(0, 1, 1)
    stage_x(1, 1)

    def chunk_pair(p_, _):
        for half in range(2):
            c = p_ * 2 + half
            xq, nxq = half, 1 - half
            for q in range(NG):
                drain(q)
                reduce_group(c, q, q)

                @pl.when(c + 1 < NCHUNK)
                def _(q=q, nxq=nxq, c=c):
                    if q == 0:
                        wait_x(nxq)
                    issue(nxq, q, q)

            @pl.when(c + 2 < NCHUNK)
            def _(c=c, xq=xq):
                stage_x(c + 2, xq)

            row0 = pl.multiple_of(base + c * CH, CH)
            pltpu.sync_copy(outv, out_hbm.at[pl.ds(row0, CH)])
        return 0

    lax.fori_loop(0, NCHUNK // 2, chunk_pair, 0)


def _pooled_logits(x_flat, len_bcast, p, b):
    mesh = plsc.VectorSubcoreMesh(core_axis_name="c", subcore_axis_name="s")
    call = pl.kernel(
        _sc_body,
        out_type=jax.ShapeDtypeStruct((B, C), jnp.float32),
        mesh=mesh,
        scratch_types=[
            pltpu.VMEM((CH * S,), jnp.int32),
            pltpu.VMEM((CH * S,), jnp.int32),
            pltpu.VMEM((GW, C), jnp.float32),
            pltpu.VMEM((GW, C), jnp.float32),
            pltpu.VMEM((CH, C), jnp.float32),
            pltpu.VMEM((RPW, C), jnp.float32),
            pltpu.VMEM((C,), jnp.float32),
            pltpu.SemaphoreType.DMA,
            pltpu.SemaphoreType.DMA,
            pltpu.SemaphoreType.DMA,
            pltpu.SemaphoreType.DMA,
        ],
        compiler_params=pltpu.CompilerParams(use_tc_tiling_on_sc=False),
    )
    return call(x_flat, len_bcast, p, b)


def kernel(x, lengths, table, W, b):
    p = _project_table(table.T, W.T).reshape(VP, C)
    x_lin = ((x >> 13) << 13) | ((x & 1023) << 3) | ((x >> 10) & 7)
    x_flat = x_lin.reshape(B * S)
    len_bcast = jnp.broadcast_to(
        lengths.astype(jnp.float32).reshape(B // 8, 8, 1),
        (B // 8, 8, C)).reshape(B // 8, 8 * C).reshape(B, C)
    return _pooled_logits(x_flat, len_bcast, p, b)


# speedup vs baseline: 7.2140x; 1.0002x over previous
import jax
import jax.numpy as jnp
from jax import lax
from jax.experimental import pallas as pl
from jax.experimental.pallas import tpu as pltpu
from jax.experimental.pallas import tpu_sc as plsc

B = 16384
S = 200
V = 1000000
D = 64
C = 16

NC, NS = 2, 16
NW = NC * NS
RPW = B // NW
CH = 32
NCHUNK = RPW // CH
G = 16
GW = G * S
NG = CH // G

MBLK = 8192
NBLK = (V + MBLK - 1) // MBLK
VP = NBLK * MBLK


def _mm_body(t_ref, w_ref, o_ref):
    res = lax.dot_general(t_ref[...].astype(jnp.bfloat16),
                          w_ref[...].astype(jnp.bfloat16),
                          (((0,), (0,)), ((), ())),
                          preferred_element_type=jnp.float32)
    for j in range(8):
        o_ref[:, j * C:(j + 1) * C] = res[j * (MBLK // 8):(j + 1) * (MBLK // 8), :]


def _project_table(table_t, w_t):
    return pl.pallas_call(
        _mm_body,
        grid=(NBLK,),
        in_specs=[
            pl.BlockSpec((D, MBLK), lambda i: (0, i)),
            pl.BlockSpec((D, C), lambda i: (0, 0)),
        ],
        out_specs=pl.BlockSpec((MBLK // 8, 8 * C), lambda i: (i, 0)),
        out_shape=jax.ShapeDtypeStruct((VP // 8, 8 * C), jnp.float32),
    )(table_t, w_t)


def _sc_body(x_hbm, len_hbm, p_hbm, b_hbm, out_hbm,
             xv0, xv1, rb0, rb1, outv, lenv, biasv,
             sr0, sr1, sx0, sx1):
    xvs = (xv0, xv1)
    rbufs = (rb0, rb1)
    sems = (sr0, sr1)
    xsems = (sx0, sx1)
    wid = lax.axis_index("c") * NS + lax.axis_index("s")
    base = wid * RPW

    pltpu.sync_copy(b_hbm, biasv)
    pltpu.sync_copy(len_hbm.at[pl.ds(base, RPW)], lenv)
    bias = biasv[...]

    def stage_x(c, xq):
        off = pl.multiple_of((base + c * CH) * S, 128)
        pltpu.async_copy(x_hbm.at[pl.ds(off, CH * S)], xvs[xq], xsems[xq])

    def wait_x(xq):
        pltpu.make_async_copy(x_hbm.at[pl.ds(0, CH * S)],
                              xvs[xq], xsems[xq]).wait()

    def issue(xq, g, q):
        off = pl.multiple_of(g * GW, 128)
        pltpu.async_copy(p_hbm.at[xvs[xq].at[pl.ds(off, GW)]],
                         rbufs[q], sems[q])

    def drain(q):
        pltpu.make_async_copy(p_hbm.at[xv0.at[pl.ds(0, GW)]],
                              rbufs[q], sems[q]).wait()

    def reduce_group(c, g, q):
        buf = rbufs[q]

        def rbody(j, _):
            tb = j * S
            a0 = buf[tb, :]
            a1 = buf[tb + 1, :]
            a2 = buf[tb + 2, :]
            a3 = buf[tb + 3, :]
            for t in range(4, S, 4):
                a0 = a0 + buf[tb + t, :]
                a1 = a1 + buf[tb + t + 1, :]
                a2 = a2 + buf[tb + t + 2, :]
                a3 = a3 + buf[tb + t + 3, :]
            s = (a0 + a1) + (a2 + a3)
            row = g * G + j
            outv[row, :] = s / lenv[c * CH + row, :] + bias
            return 0

        lax.fori_loop(0, G, rbody, 0)

    stage_x(0, 0)
    wait_x(0)
    issue(0, 0, 0)
    issue(0, 1, 1)
    stage_x(1, 1)

    def chunk_pair(p_, _):
        for half in range(2):
            c = p_ * 2 + half
            xq, nxq = half, 1 - half
            for q in range(NG):
                drain(q)
                reduce_group(c, q, q)

                @pl.when(c + 1 < NCHUNK)
                def _(q=q, nxq=nxq, c=c):
                    if q == 0:
                        wait_x(nxq)
                    issue(nxq, q, q)

            @pl.when(c + 2 < NCHUNK)
            def _(c=c, xq=xq):
                stage_x(c + 2, xq)

            row0 = pl.multiple_of(base + c * CH, CH)
            pltpu.sync_copy(outv, out_hbm.at[pl.ds(row0, CH)])
        return 0

    lax.fori_loop(0, NCHUNK // 2, chunk_pair, 0)


def _pooled_logits(x_flat, len_bcast, p, b):
    mesh = plsc.VectorSubcoreMesh(core_axis_name="c", subcore_axis_name="s")
    call = pl.kernel(
        _sc_body,
        out_type=jax.ShapeDtypeStruct((B, C), jnp.float32),
        mesh=mesh,
        scratch_types=[
            pltpu.VMEM((CH * S,), jnp.int32),
            pltpu.VMEM((CH * S,), jnp.int32),
            pltpu.VMEM((GW, C), jnp.float32),
            pltpu.VMEM((GW, C), jnp.float32),
            pltpu.VMEM((CH, C), jnp.float32),
            pltpu.VMEM((RPW, C), jnp.float32),
            pltpu.VMEM((C,), jnp.float32),
            pltpu.SemaphoreType.DMA,
            pltpu.SemaphoreType.DMA,
            pltpu.SemaphoreType.DMA,
            pltpu.SemaphoreType.DMA,
        ],
        compiler_params=pltpu.CompilerParams(use_tc_tiling_on_sc=False),
    )
    return call(x_flat, len_bcast, p, b)


def kernel(x, lengths, table, W, b):
    p = _project_table(table.T, W.T).reshape(VP, C)
    x_lin = ((x >> 13) << 13) | ((x & 1023) << 3) | ((x >> 10) & 7)
    x2d = lax.optimization_barrier(x_lin.reshape(B * S // 128, 128))
    x_flat = x2d.reshape(B * S)
    len_bcast = jnp.broadcast_to(
        lengths.astype(jnp.float32).reshape(B // 8, 8, 1),
        (B // 8, 8, C)).reshape(B // 8, 8 * C).reshape(B, C)
    return _pooled_logits(x_flat, len_bcast, p, b)


# speedup vs baseline: 7.7725x; 1.0774x over previous
import jax
import jax.numpy as jnp
from jax import lax
from jax.experimental import pallas as pl
from jax.experimental.pallas import tpu as pltpu
from jax.experimental.pallas import tpu_sc as plsc

B = 16384
S = 200
V = 1000000
D = 64
C = 16

NC, NS = 2, 16
NW = NC * NS
RPW = B // NW
CH = 32
NCHUNK = RPW // CH
G = 16
GW = G * S
NG = CH // G

MBLK = 16384
NBLK = (V + MBLK - 1) // MBLK
VP = NBLK * MBLK


def _mm_body(t_ref, w_ref, o_ref):
    res = lax.dot_general(t_ref[...].astype(jnp.bfloat16),
                          w_ref[...].astype(jnp.bfloat16),
                          (((0,), (0,)), ((), ())),
                          preferred_element_type=jnp.float32)
    for j in range(8):
        o_ref[:, j * C:(j + 1) * C] = res[j * (MBLK // 8):(j + 1) * (MBLK // 8), :]


def _project_table(table_t, w_t):
    return pl.pallas_call(
        _mm_body,
        grid=(NBLK,),
        in_specs=[
            pl.BlockSpec((D, MBLK), lambda i: (0, i)),
            pl.BlockSpec((D, C), lambda i: (0, 0)),
        ],
        out_specs=pl.BlockSpec((MBLK // 8, 8 * C), lambda i: (i, 0)),
        out_shape=jax.ShapeDtypeStruct((VP // 8, 8 * C), jnp.float32),
    )(table_t, w_t)


def _sc_body(x_hbm, len_hbm, p_hbm, b_hbm, out_hbm,
             xv0, xv1, rb0, rb1, outv, lenv, biasv,
             sr0, sr1, sx0, sx1):
    xvs = (xv0, xv1)
    rbufs = (rb0, rb1)
    sems = (sr0, sr1)
    xsems = (sx0, sx1)
    wid = lax.axis_index("c") * NS + lax.axis_index("s")
    base = wid * RPW

    pltpu.sync_copy(b_hbm, biasv)
    pltpu.sync_copy(len_hbm.at[pl.ds(base, RPW)], lenv)
    bias = biasv[...]

    def stage_x(c, xq):
        off = pl.multiple_of((base + c * CH) * S, 128)
        pltpu.async_copy(x_hbm.at[pl.ds(off, CH * S)], xvs[xq], xsems[xq])

    def wait_x(xq):
        pltpu.make_async_copy(x_hbm.at[pl.ds(0, CH * S)],
                              xvs[xq], xsems[xq]).wait()

    def issue(xq, g, q):
        off = pl.multiple_of(g * GW, 128)
        pltpu.async_copy(p_hbm.at[xvs[xq].at[pl.ds(off, GW)]],
                         rbufs[q], sems[q])

    def drain(q):
        pltpu.make_async_copy(p_hbm.at[xv0.at[pl.ds(0, GW)]],
                              rbufs[q], sems[q]).wait()

    def reduce_group(c, g, q):
        buf = rbufs[q]

        def rbody(j, _):
            tb = j * S
            a0 = buf[tb, :]
            a1 = buf[tb + 1, :]
            a2 = buf[tb + 2, :]
            a3 = buf[tb + 3, :]
            for t in range(4, S, 4):
                a0 = a0 + buf[tb + t, :]
                a1 = a1 + buf[tb + t + 1, :]
                a2 = a2 + buf[tb + t + 2, :]
                a3 = a3 + buf[tb + t + 3, :]
            s = (a0 + a1) + (a2 + a3)
            row = g * G + j
            outv[row, :] = s / lenv[c * CH + row, :] + bias
            return 0

        lax.fori_loop(0, G, rbody, 0)

    stage_x(0, 0)
    wait_x(0)
    issue(0, 0, 0)
    issue(0, 1, 1)
    stage_x(1, 1)

    def chunk_pair(p_, _):
        for half in range(2):
            c = p_ * 2 + half
            xq, nxq = half, 1 - half
            for q in range(NG):
                drain(q)
                reduce_group(c, q, q)

                @pl.when(c + 1 < NCHUNK)
                def _(q=q, nxq=nxq, c=c):
                    if q == 0:
                        wait_x(nxq)
                    issue(nxq, q, q)

            @pl.when(c + 2 < NCHUNK)
            def _(c=c, xq=xq):
                stage_x(c + 2, xq)

            row0 = pl.multiple_of(base + c * CH, CH)
            pltpu.sync_copy(outv, out_hbm.at[pl.ds(row0, CH)])
        return 0

    lax.fori_loop(0, NCHUNK // 2, chunk_pair, 0)


def _pooled_logits(x_flat, len_bcast, p, b):
    mesh = plsc.VectorSubcoreMesh(core_axis_name="c", subcore_axis_name="s")
    call = pl.kernel(
        _sc_body,
        out_type=jax.ShapeDtypeStruct((B, C), jnp.float32),
        mesh=mesh,
        scratch_types=[
            pltpu.VMEM((CH * S,), jnp.int32),
            pltpu.VMEM((CH * S,), jnp.int32),
            pltpu.VMEM((GW, C), jnp.float32),
            pltpu.VMEM((GW, C), jnp.float32),
            pltpu.VMEM((CH, C), jnp.float32),
            pltpu.VMEM((RPW, C), jnp.float32),
            pltpu.VMEM((C,), jnp.float32),
            pltpu.SemaphoreType.DMA,
            pltpu.SemaphoreType.DMA,
            pltpu.SemaphoreType.DMA,
            pltpu.SemaphoreType.DMA,
        ],
        compiler_params=pltpu.CompilerParams(use_tc_tiling_on_sc=False),
    )
    return call(x_flat, len_bcast, p, b)


def kernel(x, lengths, table, W, b):
    p = _project_table(table.T, W.T).reshape(VP, C)
    lg = MBLK.bit_length() - 1
    q = MBLK // 8
    x_lin = ((x >> lg) << lg) | ((x & (q - 1)) << 3) | ((x >> (lg - 3)) & 7)
    x2d = lax.optimization_barrier(x_lin.reshape(B * S // 128, 128))
    x_flat = x2d.reshape(B * S)
    lb2d = lax.optimization_barrier(jnp.broadcast_to(
        lengths.astype(jnp.float32).reshape(B // 8, 8, 1),
        (B // 8, 8, C)).reshape(B // 8, 8 * C))
    len_bcast = lb2d.reshape(B, C)
    return _pooled_logits(x_flat, len_bcast, p, b)
